# baseline (speedup 1.0000x reference)
import jax, jax.numpy as jnp
from jax.experimental import pallas as pl

N = 262144
M = N * 98 // 128
BM = 1024
G = M // BM

def _body(gx_ref, out_ref):
    out_ref[...] = gx_ref[0:8, :]

def kernel(logits, pre_gnn_input, gdata_x, gdata_target_vec, gdata_batch,
           Wl, Wt, Wn, Wo, Wp, bp):
    gxf = gdata_x.reshape(M, 128)
    out = pl.pallas_call(
        _body,
        grid=(G,),
        in_specs=[pl.BlockSpec((BM, 128), lambda i: (i, 0))],
        out_specs=pl.BlockSpec((8, 128), lambda i: (i, 0)),
        out_shape=jax.ShapeDtypeStruct((G * 8, 128), jnp.float32),
    )(gxf)
    return jnp.broadcast_to(jnp.sum(out), (N, 1))


# speedup vs baseline: 119.3884x; 119.3884x over previous
import functools

import jax
import jax.numpy as jnp
from jax import lax
from jax.experimental import pallas as pl
from jax.experimental.pallas import tpu as pltpu
from jax.experimental.pallas import tpu_sc as plsc

N = 262144
SEGS = 8192
CELLS = 49
NB = N // 128
GB = 32
GRID = NB // GB
EMB = 32

NC, NS, L = 2, 16, 16
NW = NC * NS
CH = N // NW
SL = SEGS // NS


def _dense_body(gx_ref, lg_ref, tv_ref, bgx_ref, blg_ref, btv_ref, bp_ref,
                bias_ref, out_ref):
    gx = gx_ref[...]
    s2 = jnp.sum(gx, axis=0)
    tvb = tv_ref[...]
    tve = jnp.concatenate([tvb, jnp.abs(tvb)], axis=0)
    lg = lg_ref[...]
    f32 = jnp.float32
    pre = jnp.dot(bgx_ref[...], s2, preferred_element_type=f32)
    pre = pre + jnp.dot(blg_ref[...], lg, preferred_element_type=f32)
    pre = pre + jnp.dot(btv_ref[...], tve, preferred_element_type=f32)
    act = jnp.maximum(pre, 0.0)
    h = jnp.dot(bp_ref[...], act, preferred_element_type=f32)
    out_ref[...] = h + bias_ref[...]


def _dense_pass(gx3, lg5, tvj, b_gx, b_lg, b_tv, b_p, bias11):
    return pl.pallas_call(
        _dense_body,
        grid=(GRID,),
        in_specs=[
            pl.BlockSpec((CELLS, 2 * GB, 128), lambda i: (0, i, 0)),
            pl.BlockSpec((5 * GB, 128), lambda i: (i, 0)),
            pl.BlockSpec((2 * GB, 128), lambda i: (i, 0)),
            pl.BlockSpec((GB * EMB, 2 * GB), lambda i: (0, 0)),
            pl.BlockSpec((GB * EMB, 5 * GB), lambda i: (0, 0)),
            pl.BlockSpec((GB * EMB, 4 * GB), lambda i: (0, 0)),
            pl.BlockSpec((GB, GB * EMB), lambda i: (0, 0)),
            pl.BlockSpec((1, 1), lambda i: (0, 0)),
        ],
        out_specs=pl.BlockSpec((GB, 128), lambda i: (i, 0)),
        out_shape=jax.ShapeDtypeStruct((NB, 128), jnp.float32),
    )(gx3, lg5, tvj, b_gx, b_lg, b_tv, b_p, bias11)


def _b1_body(h_hbm, ids_hbm, sums_hbm, cnts_hbm,
             hv, iv, ones_v, zer_v, seg_sp, cnt_sp):
    cid = lax.axis_index("c")
    sid = lax.axis_index("s")
    wid = cid * NS + sid

    def fill_zero(i, _):
        zer_v[pl.ds(i * L, L)] = jnp.zeros((L,), jnp.float32)
        return 0

    lax.fori_loop(0, SL // L, fill_zero, 0)

    def fill_one(i, _):
        ones_v[pl.ds(i * L, L)] = jnp.ones((L,), jnp.float32)
        return 0

    lax.fori_loop(0, CH // L, fill_one, 0)

    pltpu.sync_copy(zer_v, seg_sp.at[pl.ds(sid * SL, SL)])
    pltpu.sync_copy(zer_v, cnt_sp.at[pl.ds(sid * SL, SL)])
    plsc.subcore_barrier()

    pltpu.sync_copy(h_hbm.at[pl.ds(wid * CH, CH)], hv)
    pltpu.sync_copy(ids_hbm.at[pl.ds(wid * CH, CH)], iv)
    pltpu.sync_copy(hv, seg_sp.at[iv], add=True)
    pltpu.sync_copy(ones_v, cnt_sp.at[iv], add=True)
    plsc.subcore_barrier()

    @pl.when(sid == 0)
    def _():
        pltpu.sync_copy(seg_sp, sums_hbm.at[pl.ds(cid * SEGS, SEGS)])
        pltpu.sync_copy(cnt_sp, cnts_hbm.at[pl.ds(cid * SEGS, SEGS)])


def _b2_body(sums_hbm, cnts_hbm, ids_hbm, out_hbm,
             s0, s1, c0, c1, mean_sl, mean_full, iv, ov, mean_sp):
    cid = lax.axis_index("c")
    sid = lax.axis_index("s")
    wid = cid * NS + sid
    base = sid * SL

    pltpu.sync_copy(sums_hbm.at[pl.ds(base, SL)], s0)
    pltpu.sync_copy(sums_hbm.at[pl.ds(SEGS + base, SL)], s1)
    pltpu.sync_copy(cnts_hbm.at[pl.ds(base, SL)], c0)
    pltpu.sync_copy(cnts_hbm.at[pl.ds(SEGS + base, SL)], c1)

    def combine(j, _):
        sl = pl.ds(j * L, L)
        tot = s0[sl] + s1[sl]
        cnt = jnp.maximum(c0[sl] + c1[sl], 1.0)
        r = 1.0 / cnt
        r = r * (2.0 - cnt * r)
        r = r * (2.0 - cnt * r)
        mean_sl[sl] = tot * r
        return 0

    lax.fori_loop(0, SL // L, combine, 0)

    pltpu.sync_copy(mean_sl, mean_sp.at[pl.ds(base, SL)])
    plsc.subcore_barrier()
    pltpu.sync_copy(mean_sp, mean_full)

    pltpu.sync_copy(ids_hbm.at[pl.ds(wid * CH, CH)], iv)

    def gather(i, _):
        sl = pl.ds(i * L, L)
        ov[sl] = plsc.load_gather(mean_full, [iv[sl]])
        return 0

    lax.fori_loop(0, CH // L, gather, 0)

    pltpu.sync_copy(ov, out_hbm.at[pl.ds(wid * CH, CH)])


def _segment_mean_gather(h_flat, ids):
    mesh = plsc.VectorSubcoreMesh(core_axis_name="c", subcore_axis_name="s",
                                  num_cores=NC, num_subcores=NS)
    params = pltpu.CompilerParams(needs_layout_passes=False)
    sums, cnts = pl.kernel(
        _b1_body,
        out_type=(jax.ShapeDtypeStruct((NC * SEGS,), jnp.float32),
                  jax.ShapeDtypeStruct((NC * SEGS,), jnp.float32)),
        mesh=mesh,
        scratch_types=[
            pltpu.VMEM((CH,), jnp.float32),
            pltpu.VMEM((CH,), jnp.int32),
            pltpu.VMEM((CH,), jnp.float32),
            pltpu.VMEM((SL,), jnp.float32),
            pltpu.VMEM_SHARED((SEGS,), jnp.float32),
            pltpu.VMEM_SHARED((SEGS,), jnp.float32),
        ],
        compiler_params=params,
    )(h_flat, ids)

    out = pl.kernel(
        _b2_body,
        out_type=jax.ShapeDtypeStruct((N,), jnp.float32),
        mesh=mesh,
        scratch_types=[
            pltpu.VMEM((SL,), jnp.float32),
            pltpu.VMEM((SL,), jnp.float32),
            pltpu.VMEM((SL,), jnp.float32),
            pltpu.VMEM((SL,), jnp.float32),
            pltpu.VMEM((SL,), jnp.float32),
            pltpu.VMEM((SEGS,), jnp.float32),
            pltpu.VMEM((CH,), jnp.int32),
            pltpu.VMEM((CH,), jnp.float32),
            pltpu.VMEM_SHARED((SEGS,), jnp.float32),
        ],
        compiler_params=params,
    )(sums, cnts, ids)
    return out


def kernel(logits, pre_gnn_input, gdata_x, gdata_target_vec, gdata_batch,
           Wl, Wt, Wn, Wo, Wp, bp):
    gx3 = gdata_x.reshape(NB, 128, 7, 7, 2).transpose(2, 3, 0, 4, 1) \
                 .reshape(CELLS, 2 * NB, 128)
    lg5 = logits.reshape(NB, 128, 5).transpose(0, 2, 1).reshape(5 * NB, 128)
    tvj = gdata_target_vec.reshape(NB, 128, 2).transpose(0, 2, 1) \
                          .reshape(2 * NB, 128)

    eye = jnp.eye(GB, dtype=jnp.float32)
    inv_cells = jnp.float32(1.0 / 25.0)
    w2 = jnp.stack([Wo[0], Wn[0]], axis=1) * inv_cells
    b_gx = jnp.kron(eye, w2)
    b_lg = jnp.kron(eye, Wl.T)
    b_tv = jnp.concatenate(
        [jnp.kron(eye, Wt[:2].T),
         jnp.kron(eye, jnp.stack([Wt[2], Wt[2]], axis=1))], axis=1)
    b_p = jnp.kron(eye, Wp.T)
    bias11 = bp.reshape(1, 1)

    h2 = _dense_pass(gx3, lg5, tvj, b_gx, b_lg, b_tv, b_p, bias11)
    ids = gdata_batch.astype(jnp.int32)
    out = _segment_mean_gather(h2.reshape(N), ids)
    return out.reshape(N, 1)


# speedup vs baseline: 121.1817x; 1.0150x over previous
import functools

import jax
import jax.numpy as jnp
from jax import lax
from jax.experimental import pallas as pl
from jax.experimental.pallas import tpu as pltpu
from jax.experimental.pallas import tpu_sc as plsc

N = 262144
SEGS = 8192
CELLS = 49
NB = N // 128
GB = 32
GRID = NB // GB
EMB = 32

NC, NS, L = 2, 16, 16
NW = NC * NS
CH = N // NW
SL = SEGS // NS


def _dense_body(gx_ref, lg_ref, tv_ref, bgx_ref, blg_ref, btv_ref, bp_ref,
                bias_ref, out_ref):
    gx = gx_ref[...]
    s2 = jnp.sum(gx, axis=0)
    tvb = tv_ref[...]
    tve = jnp.concatenate([tvb, jnp.abs(tvb)], axis=0)
    lg = lg_ref[...]
    f32 = jnp.float32
    pre = jnp.dot(bgx_ref[...], s2, preferred_element_type=f32)
    pre = pre + jnp.dot(blg_ref[...], lg, preferred_element_type=f32)
    pre = pre + jnp.dot(btv_ref[...], tve, preferred_element_type=f32)
    act = jnp.maximum(pre, 0.0)
    h = jnp.dot(bp_ref[...], act, preferred_element_type=f32)
    out_ref[...] = h + bias_ref[...]


def _dense_pass(gx3, lg5, tvj, b_gx, b_lg, b_tv, b_p, bias11):
    return pl.pallas_call(
        _dense_body,
        grid=(GRID,),
        in_specs=[
            pl.BlockSpec((CELLS, 2 * GB, 128), lambda i: (0, i, 0)),
            pl.BlockSpec((5 * GB, 128), lambda i: (i, 0)),
            pl.BlockSpec((2 * GB, 128), lambda i: (i, 0)),
            pl.BlockSpec((GB * EMB, 2 * GB), lambda i: (0, 0)),
            pl.BlockSpec((GB * EMB, 5 * GB), lambda i: (0, 0)),
            pl.BlockSpec((GB * EMB, 4 * GB), lambda i: (0, 0)),
            pl.BlockSpec((GB, GB * EMB), lambda i: (0, 0)),
            pl.BlockSpec((1, 1), lambda i: (0, 0)),
        ],
        out_specs=pl.BlockSpec((GB, 128), lambda i: (i, 0)),
        out_shape=jax.ShapeDtypeStruct((NB, 128), jnp.float32),
    )(gx3, lg5, tvj, b_gx, b_lg, b_tv, b_p, bias11)


def _b1_body(h_hbm, ids_hbm, ones_hbm, sums_hbm, cnts_hbm,
             hv, iv, ones_v, zer_v, seg_sp, cnt_sp, sem_in, sem_sc):
    cid = lax.axis_index("c")
    sid = lax.axis_index("s")
    wid = cid * NS + sid

    ld_h = pltpu.async_copy(h_hbm.at[pl.ds(wid * CH, CH)], hv, sem_in)
    ld_i = pltpu.async_copy(ids_hbm.at[pl.ds(wid * CH, CH)], iv, sem_in)
    ld_o = pltpu.async_copy(ones_hbm.at[pl.ds(wid * CH, CH)], ones_v, sem_in)

    def fill_zero(i, _):
        zer_v[pl.ds(i * L, L)] = jnp.zeros((L,), jnp.float32)
        return 0

    lax.fori_loop(0, SL // L, fill_zero, 0)

    pltpu.sync_copy(zer_v, seg_sp.at[pl.ds(sid * SL, SL)])
    pltpu.sync_copy(zer_v, cnt_sp.at[pl.ds(sid * SL, SL)])
    ld_h.wait()
    ld_i.wait()
    ld_o.wait()
    plsc.subcore_barrier()

    sc_h = pltpu.async_copy(hv, seg_sp.at[iv], sem_sc, add=True)
    sc_o = pltpu.async_copy(ones_v, cnt_sp.at[iv], sem_sc, add=True)
    sc_h.wait()
    sc_o.wait()
    plsc.subcore_barrier()

    @pl.when(sid == 0)
    def _():
        pltpu.sync_copy(seg_sp, sums_hbm.at[pl.ds(cid * SEGS, SEGS)])
        pltpu.sync_copy(cnt_sp, cnts_hbm.at[pl.ds(cid * SEGS, SEGS)])


def _b2_body(sums_hbm, cnts_hbm, ids_hbm, out_hbm,
             s0, s1, c0, c1, mean_sl, mean_full, iv, ov, mean_sp):
    cid = lax.axis_index("c")
    sid = lax.axis_index("s")
    wid = cid * NS + sid
    base = sid * SL

    pltpu.sync_copy(sums_hbm.at[pl.ds(base, SL)], s0)
    pltpu.sync_copy(sums_hbm.at[pl.ds(SEGS + base, SL)], s1)
    pltpu.sync_copy(cnts_hbm.at[pl.ds(base, SL)], c0)
    pltpu.sync_copy(cnts_hbm.at[pl.ds(SEGS + base, SL)], c1)

    def combine(j, _):
        sl = pl.ds(j * L, L)
        tot = s0[sl] + s1[sl]
        cnt = jnp.maximum(c0[sl] + c1[sl], 1.0)
        r = 1.0 / cnt
        r = r * (2.0 - cnt * r)
        r = r * (2.0 - cnt * r)
        mean_sl[sl] = tot * r
        return 0

    lax.fori_loop(0, SL // L, combine, 0)

    pltpu.sync_copy(mean_sl, mean_sp.at[pl.ds(base, SL)])
    plsc.subcore_barrier()
    pltpu.sync_copy(mean_sp, mean_full)

    pltpu.sync_copy(ids_hbm.at[pl.ds(wid * CH, CH)], iv)

    def gather(i, _):
        sl = pl.ds(i * L, L)
        ov[sl] = plsc.load_gather(mean_full, [iv[sl]])
        return 0

    lax.fori_loop(0, CH // L, gather, 0)

    pltpu.sync_copy(ov, out_hbm.at[pl.ds(wid * CH, CH)])


def _segment_mean_gather(h_flat, ids):
    mesh = plsc.VectorSubcoreMesh(core_axis_name="c", subcore_axis_name="s",
                                  num_cores=NC, num_subcores=NS)
    params = pltpu.CompilerParams(needs_layout_passes=False)
    ones = jnp.ones((N,), jnp.float32)
    sums, cnts = pl.kernel(
        _b1_body,
        out_type=(jax.ShapeDtypeStruct((NC * SEGS,), jnp.float32),
                  jax.ShapeDtypeStruct((NC * SEGS,), jnp.float32)),
        mesh=mesh,
        scratch_types=[
            pltpu.VMEM((CH,), jnp.float32),
            pltpu.VMEM((CH,), jnp.int32),
            pltpu.VMEM((CH,), jnp.float32),
            pltpu.VMEM((SL,), jnp.float32),
            pltpu.VMEM_SHARED((SEGS,), jnp.float32),
            pltpu.VMEM_SHARED((SEGS,), jnp.float32),
            pltpu.SemaphoreType.DMA,
            pltpu.SemaphoreType.DMA,
        ],
        compiler_params=params,
    )(h_flat, ids, ones)

    out = pl.kernel(
        _b2_body,
        out_type=jax.ShapeDtypeStruct((N,), jnp.float32),
        mesh=mesh,
        scratch_types=[
            pltpu.VMEM((SL,), jnp.float32),
            pltpu.VMEM((SL,), jnp.float32),
            pltpu.VMEM((SL,), jnp.float32),
            pltpu.VMEM((SL,), jnp.float32),
            pltpu.VMEM((SL,), jnp.float32),
            pltpu.VMEM((SEGS,), jnp.float32),
            pltpu.VMEM((CH,), jnp.int32),
            pltpu.VMEM((CH,), jnp.float32),
            pltpu.VMEM_SHARED((SEGS,), jnp.float32),
        ],
        compiler_params=params,
    )(sums, cnts, ids)
    return out


def kernel(logits, pre_gnn_input, gdata_x, gdata_target_vec, gdata_batch,
           Wl, Wt, Wn, Wo, Wp, bp):
    gx3 = gdata_x.reshape(NB, 128, 7, 7, 2).transpose(2, 3, 0, 4, 1) \
                 .reshape(CELLS, 2 * NB, 128)
    lg5 = logits.reshape(NB, 128, 5).transpose(0, 2, 1).reshape(5 * NB, 128)
    tvj = gdata_target_vec.reshape(NB, 128, 2).transpose(0, 2, 1) \
                          .reshape(2 * NB, 128)

    eye = jnp.eye(GB, dtype=jnp.float32)
    inv_cells = jnp.float32(1.0 / 25.0)
    w2 = jnp.stack([Wo[0], Wn[0]], axis=1) * inv_cells
    b_gx = jnp.kron(eye, w2)
    b_lg = jnp.kron(eye, Wl.T)
    b_tv = jnp.concatenate(
        [jnp.kron(eye, Wt[:2].T),
         jnp.kron(eye, jnp.stack([Wt[2], Wt[2]], axis=1))], axis=1)
    b_p = jnp.kron(eye, Wp.T)
    bias11 = bp.reshape(1, 1)

    h2 = _dense_pass(gx3, lg5, tvj, b_gx, b_lg, b_tv, b_p, bias11)
    ids = gdata_batch.astype(jnp.int32)
    out = _segment_mean_gather(h2.reshape(N), ids)
    return out.reshape(N, 1)
